# baseline (speedup 1.0000x reference)
import functools

import jax
import jax.numpy as jnp
from jax import lax
from jax.experimental import pallas as pl
from jax.experimental.pallas import tpu as pltpu
from jax.experimental.pallas import tpu_sc as plsc

BATCH = 16384
EMBED_DIM = 32
NUM_CORES = 2
NUM_SUBCORES = 16
NUM_WORKERS = NUM_CORES * NUM_SUBCORES
CHUNK = 128
ROWS_PER_WORKER = BATCH // NUM_WORKERS
CHUNKS_PER_WORKER = ROWS_PER_WORKER // CHUNK
LANES = 16


def _combine_rows(e_ref, p_ref, o_ref):
    third = jnp.float32(1.0 / 3.0)

    @pl.loop(0, CHUNK)
    def _(r):
        for c in range(0, EMBED_DIM, LANES):
            e = e_ref[r, pl.ds(c, LANES)]
            p = p_ref[r, pl.ds(c, LANES)]
            o_ref[r, pl.ds(c, LANES)] = (e + p + e * p) * third


def _embed_kernel(
    u_idx_hbm,
    i_idx_hbm,
    u_emb_hbm,
    i_emb_hbm,
    u_prof_hbm,
    i_prof_hbm,
    u_out_hbm,
    i_out_hbm,
    u_idx_v,
    i_idx_v,
    ue_v,
    up_v,
    ie_v,
    ip_v,
    sem,
):
    wid = lax.axis_index("s") * NUM_CORES + lax.axis_index("c")
    base = wid * ROWS_PER_WORKER

    pltpu.sync_copy(u_idx_hbm.at[wid], u_idx_v)
    pltpu.sync_copy(i_idx_hbm.at[wid], i_idx_v)

    @pl.loop(0, CHUNKS_PER_WORKER)
    def _(j):
        row0 = base + j * CHUNK
        cu = pltpu.async_copy(u_emb_hbm.at[u_idx_v.at[j]], ue_v, sem)
        cp = pltpu.async_copy(u_prof_hbm.at[u_idx_v.at[j]], up_v, sem)
        ci = pltpu.async_copy(i_emb_hbm.at[i_idx_v.at[j]], ie_v, sem)
        cq = pltpu.async_copy(i_prof_hbm.at[i_idx_v.at[j]], ip_v, sem)
        cu.wait()
        cp.wait()
        ci.wait()
        cq.wait()

        _combine_rows(ue_v, up_v, ue_v)
        _combine_rows(ie_v, ip_v, ie_v)

        pltpu.sync_copy(ue_v, u_out_hbm.at[pl.ds(row0, CHUNK)])
        pltpu.sync_copy(ie_v, i_out_hbm.at[pl.ds(row0, CHUNK)])


def kernel(user_indices, item_indices, user_embedding_table,
           item_embedding_table, user_profiles, item_profiles):
    u_idx = user_indices.astype(jnp.int32).reshape(
        NUM_WORKERS, CHUNKS_PER_WORKER, CHUNK)
    i_idx = item_indices.astype(jnp.int32).reshape(
        NUM_WORKERS, CHUNKS_PER_WORKER, CHUNK)

    mesh = plsc.VectorSubcoreMesh(core_axis_name="c", subcore_axis_name="s")
    out = jax.ShapeDtypeStruct((BATCH, EMBED_DIM), jnp.float32)

    run = pl.kernel(
        _embed_kernel,
        out_type=(out, out),
        mesh=mesh,
        compiler_params=pltpu.CompilerParams(use_tc_tiling_on_sc=False),
        scratch_types=[
            pltpu.VMEM((CHUNKS_PER_WORKER, CHUNK), jnp.int32),
            pltpu.VMEM((CHUNKS_PER_WORKER, CHUNK), jnp.int32),
            pltpu.VMEM((CHUNK, EMBED_DIM), jnp.float32),
            pltpu.VMEM((CHUNK, EMBED_DIM), jnp.float32),
            pltpu.VMEM((CHUNK, EMBED_DIM), jnp.float32),
            pltpu.VMEM((CHUNK, EMBED_DIM), jnp.float32),
            pltpu.SemaphoreType.DMA,
        ],
    )
    user_features, item_features = run(
        u_idx, i_idx, user_embedding_table, item_embedding_table,
        user_profiles, item_profiles)
    return (user_features, item_features)


# speedup vs baseline: 3.4501x; 3.4501x over previous
import jax
import jax.numpy as jnp
from jax import lax
from jax.experimental import pallas as pl
from jax.experimental.pallas import tpu as pltpu
from jax.experimental.pallas import tpu_sc as plsc

BATCH = 16384
EMBED_DIM = 32
NUM_CORES = 2
NUM_SUBCORES = 16
NUM_WORKERS = NUM_CORES * NUM_SUBCORES
PER_WORKER = BATCH // NUM_WORKERS
BLK = 128
LANES = 16
GROUP = 16
NGROUPS = PER_WORKER // GROUP


def _embed_kernel(
    u_idx_hbm, i_idx_hbm,
    ue_hbm, ie_hbm, up_hbm, ip_hbm,
    u_out_hbm, i_out_hbm,
    u_idx_v, i_idx_v,
    ue_blk, up_blk, ie_blk, ip_blk,
    u_out_v, i_out_v,
    sem_ue, sem_up, sem_ie, sem_ip,
):
    w = lax.axis_index("s") * NUM_CORES + lax.axis_index("c")
    b0 = w * PER_WORKER

    pltpu.sync_copy(u_idx_hbm.at[pl.ds(b0, PER_WORKER)], u_idx_v)
    pltpu.sync_copy(i_idx_hbm.at[pl.ds(b0, PER_WORKER)], i_idx_v)

    rows_lo = lax.iota(jnp.int32, LANES)
    rows_hi = rows_lo + LANES
    third = jnp.float32(1.0 / 3.0)

    def fire(u, i, slot):
        ub = (u >> 7) * BLK
        ib = (i >> 7) * BLK
        pltpu.async_copy(ue_hbm.at[:, pl.ds(ub, BLK)], ue_blk.at[slot], sem_ue)
        pltpu.async_copy(up_hbm.at[:, pl.ds(ub, BLK)], up_blk.at[slot], sem_up)
        pltpu.async_copy(ie_hbm.at[:, pl.ds(ib, BLK)], ie_blk.at[slot], sem_ie)
        pltpu.async_copy(ip_hbm.at[:, pl.ds(ib, BLK)], ip_blk.at[slot], sem_ip)

    def drain(slot):
        pltpu.make_async_copy(ue_hbm.at[:, pl.ds(0, BLK)], ue_blk.at[slot], sem_ue).wait()
        pltpu.make_async_copy(up_hbm.at[:, pl.ds(0, BLK)], up_blk.at[slot], sem_up).wait()
        pltpu.make_async_copy(ie_hbm.at[:, pl.ds(0, BLK)], ie_blk.at[slot], sem_ie).wait()
        pltpu.make_async_copy(ip_hbm.at[:, pl.ds(0, BLK)], ip_blk.at[slot], sem_ip).wait()

    def combine_col(e_blk, p_blk, out_v, lane_vec, col):
        col_v = jnp.full((LANES,), col, jnp.int32)
        e_lo = plsc.load_gather(e_blk, [rows_lo, lane_vec])
        e_hi = plsc.load_gather(e_blk, [rows_hi, lane_vec])
        p_lo = plsc.load_gather(p_blk, [rows_lo, lane_vec])
        p_hi = plsc.load_gather(p_blk, [rows_hi, lane_vec])
        o_lo = (e_lo + p_lo + e_lo * p_lo) * third
        o_hi = (e_hi + p_hi + e_hi * p_hi) * third
        plsc.store_scatter(out_v, [rows_lo, col_v], o_lo)
        plsc.store_scatter(out_v, [rows_hi, col_v], o_hi)

    def process(u, i, slot, col):
        u_lane = jnp.full((LANES,), u & (BLK - 1), jnp.int32)
        i_lane = jnp.full((LANES,), i & (BLK - 1), jnp.int32)
        combine_col(ue_blk.at[slot], up_blk.at[slot], u_out_v, u_lane, col)
        combine_col(ie_blk.at[slot], ip_blk.at[slot], i_out_v, i_lane, col)

    uvec0 = u_idx_v[pl.ds(0, GROUP)]
    ivec0 = i_idx_v[pl.ds(0, GROUP)]
    fire(uvec0[0], ivec0[0], 0)

    @pl.loop(0, NGROUPS)
    def _(g):
        uvec = u_idx_v[pl.ds(g * GROUP, GROUP)]
        ivec = i_idx_v[pl.ds(g * GROUP, GROUP)]
        un = u_idx_v[pl.ds((g + 1) * GROUP % PER_WORKER, GROUP)]
        in_ = i_idx_v[pl.ds((g + 1) * GROUP % PER_WORKER, GROUP)]
        for k in range(GROUP):
            slot = k & 1
            if k < GROUP - 1:
                fire(uvec[k + 1], ivec[k + 1], 1 - slot)
            else:
                fire(un[0], in_[0], 1 - slot)
            drain(slot)
            process(uvec[k], ivec[k], slot, g * GROUP + k)

    drain(0)

    pltpu.sync_copy(u_out_v, u_out_hbm.at[:, pl.ds(b0, PER_WORKER)])
    pltpu.sync_copy(i_out_v, i_out_hbm.at[:, pl.ds(b0, PER_WORKER)])


def kernel(user_indices, item_indices, user_embedding_table,
           item_embedding_table, user_profiles, item_profiles):
    u_idx = user_indices.astype(jnp.int32)
    i_idx = item_indices.astype(jnp.int32)

    mesh = plsc.VectorSubcoreMesh(core_axis_name="c", subcore_axis_name="s")
    out_t = jax.ShapeDtypeStruct((EMBED_DIM, BATCH), jnp.float32)
    blk = pltpu.VMEM((2, EMBED_DIM, BLK), jnp.float32)

    run = pl.kernel(
        _embed_kernel,
        out_type=(out_t, out_t),
        mesh=mesh,
        compiler_params=pltpu.CompilerParams(needs_layout_passes=False),
        scratch_types=[
            pltpu.VMEM((PER_WORKER,), jnp.int32),
            pltpu.VMEM((PER_WORKER,), jnp.int32),
            blk, blk, blk, blk,
            pltpu.VMEM((EMBED_DIM, PER_WORKER), jnp.float32),
            pltpu.VMEM((EMBED_DIM, PER_WORKER), jnp.float32),
            pltpu.SemaphoreType.DMA,
            pltpu.SemaphoreType.DMA,
            pltpu.SemaphoreType.DMA,
            pltpu.SemaphoreType.DMA,
        ],
    )
    u_out_t, i_out_t = run(
        u_idx, i_idx,
        user_embedding_table.T, item_embedding_table.T,
        user_profiles.T, item_profiles.T)
    return (u_out_t.T, i_out_t.T)


# speedup vs baseline: 4.1682x; 1.2081x over previous
import jax
import jax.numpy as jnp
from jax import lax
from jax.experimental import pallas as pl
from jax.experimental.pallas import tpu as pltpu
from jax.experimental.pallas import tpu_sc as plsc

BATCH = 16384
EMBED_DIM = 32
NUM_CORES = 2
NUM_SUBCORES = 16
NUM_WORKERS = NUM_CORES * NUM_SUBCORES
PER_WORKER = BATCH // NUM_WORKERS
BLK = 128
LANES = 16
GROUP = 16
NGROUPS = PER_WORKER // GROUP
NSLOT = 4


def _embed_kernel(
    u_idx_hbm, i_idx_hbm,
    ue_hbm, ie_hbm, up_hbm, ip_hbm,
    u_out_hbm, i_out_hbm,
    u_idx_v, i_idx_v,
    ue_blk, up_blk, ie_blk, ip_blk,
    u_out_v, i_out_v,
    sem_ue, sem_up, sem_ie, sem_ip,
):
    w = lax.axis_index("s") * NUM_CORES + lax.axis_index("c")
    b0 = w * PER_WORKER

    pltpu.sync_copy(u_idx_hbm.at[pl.ds(b0, PER_WORKER)], u_idx_v)
    pltpu.sync_copy(i_idx_hbm.at[pl.ds(b0, PER_WORKER)], i_idx_v)

    rows_lo = lax.iota(jnp.int32, LANES)
    rows_hi = rows_lo + LANES
    third = jnp.float32(1.0 / 3.0)

    def fire(u, i, slot):
        ub = (u >> 7) * BLK
        ib = (i >> 7) * BLK
        pltpu.async_copy(ue_hbm.at[:, pl.ds(ub, BLK)], ue_blk.at[slot], sem_ue)
        pltpu.async_copy(up_hbm.at[:, pl.ds(ub, BLK)], up_blk.at[slot], sem_up)
        pltpu.async_copy(ie_hbm.at[:, pl.ds(ib, BLK)], ie_blk.at[slot], sem_ie)
        pltpu.async_copy(ip_hbm.at[:, pl.ds(ib, BLK)], ip_blk.at[slot], sem_ip)

    def drain(slot):
        pltpu.make_async_copy(ue_hbm.at[:, pl.ds(0, BLK)], ue_blk.at[slot], sem_ue).wait()
        pltpu.make_async_copy(up_hbm.at[:, pl.ds(0, BLK)], up_blk.at[slot], sem_up).wait()
        pltpu.make_async_copy(ie_hbm.at[:, pl.ds(0, BLK)], ie_blk.at[slot], sem_ie).wait()
        pltpu.make_async_copy(ip_hbm.at[:, pl.ds(0, BLK)], ip_blk.at[slot], sem_ip).wait()

    def combine_col(e_blk, p_blk, out_v, lane_vec, col):
        col_v = jnp.full((LANES,), col, jnp.int32)
        e_lo = plsc.load_gather(e_blk, [rows_lo, lane_vec])
        e_hi = plsc.load_gather(e_blk, [rows_hi, lane_vec])
        p_lo = plsc.load_gather(p_blk, [rows_lo, lane_vec])
        p_hi = plsc.load_gather(p_blk, [rows_hi, lane_vec])
        o_lo = (e_lo + p_lo + e_lo * p_lo) * third
        o_hi = (e_hi + p_hi + e_hi * p_hi) * third
        plsc.store_scatter(out_v, [rows_lo, col_v], o_lo)
        plsc.store_scatter(out_v, [rows_hi, col_v], o_hi)

    def process(u, i, slot, col):
        u_lane = jnp.full((LANES,), u & (BLK - 1), jnp.int32)
        i_lane = jnp.full((LANES,), i & (BLK - 1), jnp.int32)
        combine_col(ue_blk.at[slot], up_blk.at[slot], u_out_v, u_lane, col)
        combine_col(ie_blk.at[slot], ip_blk.at[slot], i_out_v, i_lane, col)

    uvec0 = u_idx_v[pl.ds(0, GROUP)]
    ivec0 = i_idx_v[pl.ds(0, GROUP)]
    for k in range(NSLOT - 1):
        fire(uvec0[k], ivec0[k], k)

    @pl.loop(0, NGROUPS)
    def _(g):
        uvec = u_idx_v[pl.ds(g * GROUP, GROUP)]
        ivec = i_idx_v[pl.ds(g * GROUP, GROUP)]
        un = u_idx_v[pl.ds((g + 1) * GROUP % PER_WORKER, GROUP)]
        in_ = i_idx_v[pl.ds((g + 1) * GROUP % PER_WORKER, GROUP)]
        for k in range(GROUP):
            ka = k + NSLOT - 1
            if ka < GROUP:
                fire(uvec[ka], ivec[ka], ka & (NSLOT - 1))
            else:
                fire(un[ka - GROUP], in_[ka - GROUP], ka & (NSLOT - 1))
            drain(k & (NSLOT - 1))
            process(uvec[k], ivec[k], k & (NSLOT - 1), g * GROUP + k)

    for k in range(NSLOT - 1):
        drain(k)

    pltpu.sync_copy(u_out_v, u_out_hbm.at[:, pl.ds(b0, PER_WORKER)])
    pltpu.sync_copy(i_out_v, i_out_hbm.at[:, pl.ds(b0, PER_WORKER)])


def kernel(user_indices, item_indices, user_embedding_table,
           item_embedding_table, user_profiles, item_profiles):
    u_idx = user_indices.astype(jnp.int32)
    i_idx = item_indices.astype(jnp.int32)

    mesh = plsc.VectorSubcoreMesh(core_axis_name="c", subcore_axis_name="s")
    out_t = jax.ShapeDtypeStruct((EMBED_DIM, BATCH), jnp.float32)
    blk = pltpu.VMEM((NSLOT, EMBED_DIM, BLK), jnp.float32)

    run = pl.kernel(
        _embed_kernel,
        out_type=(out_t, out_t),
        mesh=mesh,
        compiler_params=pltpu.CompilerParams(needs_layout_passes=False),
        scratch_types=[
            pltpu.VMEM((PER_WORKER,), jnp.int32),
            pltpu.VMEM((PER_WORKER,), jnp.int32),
            blk, blk, blk, blk,
            pltpu.VMEM((EMBED_DIM, PER_WORKER), jnp.float32),
            pltpu.VMEM((EMBED_DIM, PER_WORKER), jnp.float32),
            pltpu.SemaphoreType.DMA,
            pltpu.SemaphoreType.DMA,
            pltpu.SemaphoreType.DMA,
            pltpu.SemaphoreType.DMA,
        ],
    )
    u_out_t, i_out_t = run(
        u_idx, i_idx,
        user_embedding_table.T, item_embedding_table.T,
        user_profiles.T, item_profiles.T)
    return (u_out_t.T, i_out_t.T)


# speedup vs baseline: 4.3685x; 1.0481x over previous
import jax
import jax.numpy as jnp
from jax import lax
from jax.experimental import pallas as pl
from jax.experimental.pallas import tpu as pltpu
from jax.experimental.pallas import tpu_sc as plsc

BATCH = 16384
EMBED_DIM = 32
NUM_CORES = 2
NUM_SUBCORES = 16
NUM_WORKERS = NUM_CORES * NUM_SUBCORES
SIDE_WORKERS = NUM_WORKERS // 2
PER_WORKER = BATCH // SIDE_WORKERS
BLK = 128
LANES = 16
GROUP = 16
NGROUPS = PER_WORKER // GROUP
NSLOT = 8


def _embed_kernel(
    u_idx_hbm, i_idx_hbm,
    ue_hbm, ie_hbm, up_hbm, ip_hbm,
    u_out_hbm, i_out_hbm,
    idx_v, e_blk, p_blk, out_v,
    sem_e, sem_p,
):
    w = lax.axis_index("s") * NUM_CORES + lax.axis_index("c")

    rows_lo = lax.iota(jnp.int32, LANES)
    rows_hi = rows_lo + LANES
    third = jnp.float32(1.0 / 3.0)

    def side_worker(idx_hbm, e_hbm, p_hbm, out_hbm, lw):
        b0 = lw * PER_WORKER
        pltpu.sync_copy(idx_hbm.at[pl.ds(b0, PER_WORKER)], idx_v)

        def fire(u, slot):
            ub = (u >> 7) * BLK
            pltpu.async_copy(e_hbm.at[:, pl.ds(ub, BLK)], e_blk.at[slot], sem_e)
            pltpu.async_copy(p_hbm.at[:, pl.ds(ub, BLK)], p_blk.at[slot], sem_p)

        def drain(slot):
            pltpu.make_async_copy(e_hbm.at[:, pl.ds(0, BLK)], e_blk.at[slot], sem_e).wait()
            pltpu.make_async_copy(p_hbm.at[:, pl.ds(0, BLK)], p_blk.at[slot], sem_p).wait()

        def process(u, slot, col):
            lane_vec = jnp.full((LANES,), u & (BLK - 1), jnp.int32)
            col_v = jnp.full((LANES,), col, jnp.int32)
            e_lo = plsc.load_gather(e_blk.at[slot], [rows_lo, lane_vec])
            e_hi = plsc.load_gather(e_blk.at[slot], [rows_hi, lane_vec])
            p_lo = plsc.load_gather(p_blk.at[slot], [rows_lo, lane_vec])
            p_hi = plsc.load_gather(p_blk.at[slot], [rows_hi, lane_vec])
            o_lo = (e_lo + p_lo + e_lo * p_lo) * third
            o_hi = (e_hi + p_hi + e_hi * p_hi) * third
            plsc.store_scatter(out_v, [rows_lo, col_v], o_lo)
            plsc.store_scatter(out_v, [rows_hi, col_v], o_hi)

        vec0 = idx_v[pl.ds(0, GROUP)]
        for k in range(NSLOT - 1):
            fire(vec0[k], k)

        @pl.loop(0, NGROUPS)
        def _(g):
            vec = idx_v[pl.ds(g * GROUP, GROUP)]
            nxt = idx_v[pl.ds((g + 1) * GROUP % PER_WORKER, GROUP)]
            for k in range(GROUP):
                ka = k + NSLOT - 1
                if ka < GROUP:
                    fire(vec[ka], ka & (NSLOT - 1))
                else:
                    fire(nxt[ka - GROUP], ka & (NSLOT - 1))
                drain(k & (NSLOT - 1))
                process(vec[k], k & (NSLOT - 1), g * GROUP + k)

        for k in range(NSLOT - 1):
            drain(k)

        pltpu.sync_copy(out_v, out_hbm.at[:, pl.ds(b0, PER_WORKER)])

    @pl.when(w < SIDE_WORKERS)
    def _():
        side_worker(u_idx_hbm, ue_hbm, up_hbm, u_out_hbm, w)

    @pl.when(w >= SIDE_WORKERS)
    def _():
        side_worker(i_idx_hbm, ie_hbm, ip_hbm, i_out_hbm, w - SIDE_WORKERS)


def kernel(user_indices, item_indices, user_embedding_table,
           item_embedding_table, user_profiles, item_profiles):
    u_idx = user_indices.astype(jnp.int32)
    i_idx = item_indices.astype(jnp.int32)

    mesh = plsc.VectorSubcoreMesh(core_axis_name="c", subcore_axis_name="s")
    out_t = jax.ShapeDtypeStruct((EMBED_DIM, BATCH), jnp.float32)
    blk = pltpu.VMEM((NSLOT, EMBED_DIM, BLK), jnp.float32)

    run = pl.kernel(
        _embed_kernel,
        out_type=(out_t, out_t),
        mesh=mesh,
        compiler_params=pltpu.CompilerParams(needs_layout_passes=False),
        scratch_types=[
            pltpu.VMEM((PER_WORKER,), jnp.int32),
            blk, blk,
            pltpu.VMEM((EMBED_DIM, PER_WORKER), jnp.float32),
            pltpu.SemaphoreType.DMA,
            pltpu.SemaphoreType.DMA,
        ],
    )
    u_out_t, i_out_t = run(
        u_idx, i_idx,
        user_embedding_table.T, item_embedding_table.T,
        user_profiles.T, item_profiles.T)
    return (u_out_t.T, i_out_t.T)
